# baseline (speedup 1.0000x reference)
import functools

import jax
import jax.numpy as jnp
from jax import lax
from jax.experimental import pallas as pl
from jax.experimental.pallas import tpu as pltpu
from jax.experimental.pallas import tpu_sc as plsc

N_NODES = 10000
N_EDGES = 320000
D_IN = 128
D_HID = 256
D_OUT = 256

NC = 2
NS = 16
ECH = 128
GROUPS = 8
GCH = 20
CHUNKS = GROUPS * GCH
EPT = CHUNKS * ECH
EPAD = NS * EPT
ROWS_PT = 632
ACC_ROWS = NS * ROWS_PT
NPAD_DEG = 10240
DEG_PT = NPAD_DEG // NS
RB = 512
GRID_R = NPAD_DEG // RB

_f32 = jnp.float32


def _sc_deg_body(edge_hbm, deg_hbm, dst_v, zbuf, ones, acc):
    cid = lax.axis_index("c")
    sid = lax.axis_index("s")
    for i in range(DEG_PT // 16):
        zbuf[pl.ds(i * 16, 16)] = jnp.zeros((16,), _f32)
    for i in range(ECH // 16):
        ones[pl.ds(i * 16, 16)] = jnp.ones((16,), _f32)
    pltpu.sync_copy(zbuf, acc.at[pl.ds(sid * DEG_PT, DEG_PT)])
    plsc.subcore_barrier()

    def group(g, carry):
        pltpu.sync_copy(edge_hbm.at[cid, sid, g, 1], dst_v)

        def body(j, c):
            pltpu.sync_copy(ones, acc.at[dst_v.at[j]], add=True)
            return c

        return lax.fori_loop(0, GCH, body, carry)

    lax.fori_loop(0, GROUPS, group, 0)
    plsc.subcore_barrier()
    pltpu.sync_copy(acc.at[pl.ds(sid * DEG_PT, DEG_PT)],
                    deg_hbm.at[cid, pl.ds(sid * DEG_PT, DEG_PT)])


def _make_sc_prop(n_half):
    def body_fn(tbl_hbm, edge_hbm, out_hbm, idx_v,
                rows0, rows1, acc, sem0, sem1, ssem0, ssem1):
        cid = lax.axis_index("c")
        sid = lax.axis_index("s")
        rows = (rows0, rows1)
        sems = (sem0, sem1)
        ssems = (ssem0, ssem1)
        for h in range(n_half):
            pltpu.sync_copy(tbl_hbm.at[cid, h, pl.ds(sid * ROWS_PT, ROWS_PT)],
                            acc.at[pl.ds(sid * ROWS_PT, ROWS_PT)])
            plsc.subcore_barrier()

            def group(g, carry):
                pltpu.sync_copy(edge_hbm.at[cid, sid, g], idx_v)
                gd = [None, None]
                sd = [None, None]
                gd[0] = pltpu.async_copy(
                    tbl_hbm.at[cid, h].at[idx_v.at[0, 0]], rows[0], sems[0])
                for j in range(GCH):
                    p = j & 1
                    q = (j + 1) & 1
                    if j + 1 < GCH:
                        if sd[q] is not None:
                            sd[q].wait()
                        gd[q] = pltpu.async_copy(
                            tbl_hbm.at[cid, h].at[idx_v.at[0, j + 1]], rows[q], sems[q])
                    gd[p].wait()
                    sd[p] = pltpu.async_copy(
                        rows[p], acc.at[idx_v.at[1, j]], ssems[p], add=True)
                sd[0].wait()
                sd[1].wait()
                return carry

            lax.fori_loop(0, GROUPS, group, 0)
            plsc.subcore_barrier()
            pltpu.sync_copy(acc.at[pl.ds(sid * ROWS_PT, ROWS_PT)],
                            out_hbm.at[cid, h, pl.ds(sid * ROWS_PT, ROWS_PT)])
            if h + 1 < n_half:
                plsc.subcore_barrier()
    return body_fn


def _sc_mesh():
    return plsc.VectorSubcoreMesh(core_axis_name="c", subcore_axis_name="s")


def _sc_deg(edges):
    return pl.kernel(
        _sc_deg_body,
        out_type=jax.ShapeDtypeStruct((NC, NPAD_DEG), _f32),
        mesh=_sc_mesh(),
        scratch_types=[
            pltpu.VMEM((GCH, ECH), jnp.int32),
            pltpu.VMEM((DEG_PT,), _f32),
            pltpu.VMEM((ECH,), _f32),
            pltpu.VMEM_SHARED((NPAD_DEG,), _f32),
        ],
    )(edges)


def _sc_prop(tbl, edges, n_half):
    return pl.kernel(
        _make_sc_prop(n_half),
        out_type=jax.ShapeDtypeStruct((NC, n_half, NPAD_DEG, D_IN), _f32),
        mesh=_sc_mesh(),
        scratch_types=[
            pltpu.VMEM((2, GCH, ECH), jnp.int32),
            pltpu.VMEM((ECH, D_IN), _f32),
            pltpu.VMEM((ECH, D_IN), _f32),
            pltpu.VMEM_SHARED((ACC_ROWS, D_IN), _f32),
            pltpu.SemaphoreType.DMA,
            pltpu.SemaphoreType.DMA,
            pltpu.SemaphoreType.DMA,
            pltpu.SemaphoreType.DMA,
        ],
    )(tbl, edges)


def _tc_weights_body(wi0, wn0, wi1, wn1, wf, bi0, bn0, bi1, bn1, bf,
                     wa0, wb0, wa1, wb1, b0, b1):
    wft = wf[pl.ds(0, D_OUT), :]
    wfb = wf[pl.ds(D_OUT, D_OUT), :]
    dot = functools.partial(jnp.dot, preferred_element_type=_f32)
    wa0[...] = dot(wi0[...], wft)
    wb0[...] = dot(wn0[...], wfb)
    wa1[...] = dot(wi1[...], wft)
    wb1[...] = dot(wn1[...], wfb)
    b0[...] = dot(bi0[...], wft) + dot(bn0[...], wfb) + bf[...]
    b1[...] = dot(bi1[...], wft) + dot(bn1[...], wfb) + bf[...]


def _tc_weights(wi0, wn0, wi1, wn1, wf, bi0, bn0, bi1, bn1, bf):
    outs = [
        jax.ShapeDtypeStruct((D_IN, D_OUT), _f32),
        jax.ShapeDtypeStruct((D_IN, D_OUT), _f32),
        jax.ShapeDtypeStruct((D_HID, D_OUT), _f32),
        jax.ShapeDtypeStruct((D_HID, D_OUT), _f32),
        jax.ShapeDtypeStruct((1, D_OUT), _f32),
        jax.ShapeDtypeStruct((1, D_OUT), _f32),
    ]
    return pl.pallas_call(_tc_weights_body, out_shape=outs)(
        wi0, wn0, wi1, wn1, wf, bi0, bn0, bi1, bn1, bf)


def _tc_prep_g_body(deg_ref, x_ref, dinv_ref, g0_ref):
    dv = lax.rsqrt(deg_ref[...] + 1.0)
    dinv_ref[...] = dv
    xv = x_ref[...]
    g0_ref[...] = jnp.stack([xv * dv[0], xv * dv[1]])[:, None]


def _tc_prep_g(deg3, x_pad):
    outs = [
        jax.ShapeDtypeStruct((NC, NPAD_DEG, 1), _f32),
        jax.ShapeDtypeStruct((NC, 1, NPAD_DEG, D_IN), _f32),
    ]
    return pl.pallas_call(
        _tc_prep_g_body,
        grid=(GRID_R,),
        in_specs=[
            pl.BlockSpec((NC, RB, 1), lambda r: (0, r, 0)),
            pl.BlockSpec((RB, D_IN), lambda r: (r, 0)),
        ],
        out_specs=[
            pl.BlockSpec((NC, RB, 1), lambda r: (0, r, 0)),
            pl.BlockSpec((NC, 1, RB, D_IN), lambda r: (0, 0, r, 0)),
        ],
        out_shape=outs,
    )(deg3, x_pad)


def _tc_dense1_body(s0_ref, dinv_ref, wa_ref, wb_ref, b_ref, g1_ref):
    s = s0_ref[...]
    dv = dinv_ref[...]
    pa = s[0, 0] * dv[0]
    pb = s[1, 0] * dv[1]
    dot = functools.partial(jnp.dot, preferred_element_type=_f32)
    t = jnp.maximum(dot(pa, wa_ref[...]) + dot(pb, wb_ref[...]) + b_ref[...], 0.0)
    ga = t * dv[0]
    gb = t * dv[1]
    g1_ref[...] = jnp.stack([
        jnp.stack([ga[:, :D_IN], ga[:, D_IN:]]),
        jnp.stack([gb[:, :D_IN], gb[:, D_IN:]]),
    ])


def _tc_dense1(s0, dinv, wa0, wb0, b0):
    return pl.pallas_call(
        _tc_dense1_body,
        grid=(GRID_R,),
        in_specs=[
            pl.BlockSpec((NC, 1, RB, D_IN), lambda r: (0, 0, r, 0)),
            pl.BlockSpec((NC, RB, 1), lambda r: (0, r, 0)),
            pl.BlockSpec((D_IN, D_OUT), lambda r: (0, 0)),
            pl.BlockSpec((D_IN, D_OUT), lambda r: (0, 0)),
            pl.BlockSpec((1, D_OUT), lambda r: (0, 0)),
        ],
        out_specs=pl.BlockSpec((NC, 2, RB, D_IN), lambda r: (0, 0, r, 0)),
        out_shape=jax.ShapeDtypeStruct((NC, 2, NPAD_DEG, D_IN), _f32),
    )(s0, dinv, wa0, wb0, b0)


def _tc_dense2_body(s1_ref, dinv_ref, wa_ref, wb_ref, b_ref, out_ref):
    s = s1_ref[...]
    dv = dinv_ref[...]
    pa = jnp.concatenate([s[0, 0], s[0, 1]], axis=1) * dv[0]
    pb = jnp.concatenate([s[1, 0], s[1, 1]], axis=1) * dv[1]
    dot = functools.partial(jnp.dot, preferred_element_type=_f32)
    z = jnp.maximum(dot(pa, wa_ref[...]) + dot(pb, wb_ref[...]) + b_ref[...], 0.0)
    r = pl.program_id(0)
    rows = r * RB + lax.broadcasted_iota(jnp.int32, (RB, 1), 0)
    zm = jnp.where(rows < N_NODES, z, 0.0)
    part = jnp.sum(zm, axis=0, keepdims=True)

    @pl.when(r == 0)
    def _():
        out_ref[...] = jnp.zeros_like(out_ref)

    out_ref[...] += part


def _tc_dense2(s1, dinv, wa1, wb1, b1):
    return pl.pallas_call(
        _tc_dense2_body,
        grid=(GRID_R,),
        in_specs=[
            pl.BlockSpec((NC, 2, RB, D_IN), lambda r: (0, 0, r, 0)),
            pl.BlockSpec((NC, RB, 1), lambda r: (0, r, 0)),
            pl.BlockSpec((D_HID, D_OUT), lambda r: (0, 0)),
            pl.BlockSpec((D_HID, D_OUT), lambda r: (0, 0)),
            pl.BlockSpec((1, D_OUT), lambda r: (0, 0)),
        ],
        out_specs=pl.BlockSpec((1, D_OUT), lambda r: (0, 0)),
        out_shape=jax.ShapeDtypeStruct((1, D_OUT), _f32),
    )(s1, dinv, wa1, wb1, b1)


def _pad_edges(ei):
    src = ei[0].astype(jnp.int32)
    dst = ei[1].astype(jnp.int32)
    pad = EPAD - N_EDGES
    src = jnp.concatenate([src, jnp.zeros((pad,), jnp.int32)])
    dst = jnp.concatenate([dst, jnp.full((pad,), N_NODES, jnp.int32)])
    both = jnp.stack([src.reshape(NS, GROUPS, GCH, ECH),
                      dst.reshape(NS, GROUPS, GCH, ECH)], axis=2)
    return both


def kernel(x, intra_edge_index, inter_edge_index,
           W_intra0, b_intra0, W_inter0, b_inter0,
           W_intra1, b_intra1, W_inter1, b_inter1,
           W_fus, b_fus):
    edges = jnp.stack([_pad_edges(intra_edge_index),
                       _pad_edges(inter_edge_index)])

    deg = _sc_deg(edges)
    deg3 = deg[:, :, None]

    x_pad = jnp.concatenate(
        [x, jnp.zeros((NPAD_DEG - N_NODES, D_IN), _f32)], axis=0)

    wa0, wb0, wa1, wb1, b0, b1 = _tc_weights(
        W_intra0, W_inter0, W_intra1, W_inter1, W_fus,
        b_intra0[None, :], b_inter0[None, :],
        b_intra1[None, :], b_inter1[None, :], b_fus[None, :])

    dinv, g0 = _tc_prep_g(deg3, x_pad)

    s0 = _sc_prop(g0, edges, 1)
    g1 = _tc_dense1(s0, dinv, wa0, wb0, b0)
    s1 = _sc_prop(g1, edges, 2)
    return _tc_dense2(s1, dinv, wa1, wb1, b1)


# speedup vs baseline: 1.0597x; 1.0597x over previous
import functools

import jax
import jax.numpy as jnp
from jax import lax
from jax.experimental import pallas as pl
from jax.experimental.pallas import tpu as pltpu
from jax.experimental.pallas import tpu_sc as plsc

N_NODES = 10000
N_EDGES = 320000
D_IN = 128
D_HID = 256
D_OUT = 256

NC = 2
NS = 16
ECH = 128
GROUPS = 8
GCH = 20
CHUNKS = GROUPS * GCH
EPT = CHUNKS * ECH
EPAD = NS * EPT
ROWS_PT = 632
ACC_ROWS = NS * ROWS_PT
NPAD_DEG = 10240
DEG_PT = NPAD_DEG // NS
RB = 512
GRID_R = NPAD_DEG // RB

_f32 = jnp.float32


def _sc_deg_body(dst_hbm, deg_hbm, dst_v, zbuf, ones, acc):
    cid = lax.axis_index("c")
    sid = lax.axis_index("s")
    for i in range(DEG_PT // 16):
        zbuf[pl.ds(i * 16, 16)] = jnp.zeros((16,), _f32)
    for i in range(ECH // 16):
        ones[pl.ds(i * 16, 16)] = jnp.ones((16,), _f32)
    pltpu.sync_copy(zbuf, acc.at[pl.ds(sid * DEG_PT, DEG_PT)])
    plsc.subcore_barrier()

    def group(g, carry):
        pltpu.sync_copy(dst_hbm.at[cid, sid, g], dst_v)

        def body(j, c):
            pltpu.sync_copy(ones, acc.at[dst_v.at[j]], add=True)
            return c

        return lax.fori_loop(0, GCH, body, carry)

    lax.fori_loop(0, GROUPS, group, 0)
    plsc.subcore_barrier()
    pltpu.sync_copy(acc.at[pl.ds(sid * DEG_PT, DEG_PT)],
                    deg_hbm.at[cid, pl.ds(sid * DEG_PT, DEG_PT)])


def _make_sc_prop(n_half):
    def body_fn(tbl_hbm, src_hbm, dst_hbm, out_hbm, src_v, dst_v,
                rows0, rows1, acc, sem0, sem1, ssem0, ssem1):
        cid = lax.axis_index("c")
        sid = lax.axis_index("s")
        rows = (rows0, rows1)
        sems = (sem0, sem1)
        ssems = (ssem0, ssem1)
        for h in range(n_half):
            pltpu.sync_copy(tbl_hbm.at[cid, h, pl.ds(sid * ROWS_PT, ROWS_PT)],
                            acc.at[pl.ds(sid * ROWS_PT, ROWS_PT)])
            plsc.subcore_barrier()

            def group(g, carry):
                pltpu.sync_copy(src_hbm.at[cid, sid, g], src_v)
                pltpu.sync_copy(dst_hbm.at[cid, sid, g], dst_v)
                gd = [None, None]
                sd = [None, None]
                gd[0] = pltpu.async_copy(
                    tbl_hbm.at[cid, h].at[src_v.at[0]], rows[0], sems[0])
                for j in range(GCH):
                    p = j & 1
                    q = (j + 1) & 1
                    if j + 1 < GCH:
                        if sd[q] is not None:
                            sd[q].wait()
                        gd[q] = pltpu.async_copy(
                            tbl_hbm.at[cid, h].at[src_v.at[j + 1]], rows[q], sems[q])
                    gd[p].wait()
                    sd[p] = pltpu.async_copy(
                        rows[p], acc.at[dst_v.at[j]], ssems[p], add=True)
                sd[0].wait()
                sd[1].wait()
                return carry

            lax.fori_loop(0, GROUPS, group, 0)
            plsc.subcore_barrier()
            pltpu.sync_copy(acc.at[pl.ds(sid * ROWS_PT, ROWS_PT)],
                            out_hbm.at[cid, h, pl.ds(sid * ROWS_PT, ROWS_PT)])
            if h + 1 < n_half:
                plsc.subcore_barrier()
    return body_fn


def _sc_mesh():
    return plsc.VectorSubcoreMesh(core_axis_name="c", subcore_axis_name="s")


def _sc_deg(dst2):
    return pl.kernel(
        _sc_deg_body,
        out_type=jax.ShapeDtypeStruct((NC, NPAD_DEG), _f32),
        mesh=_sc_mesh(),
        scratch_types=[
            pltpu.VMEM((GCH, ECH), jnp.int32),
            pltpu.VMEM((DEG_PT,), _f32),
            pltpu.VMEM((ECH,), _f32),
            pltpu.VMEM_SHARED((NPAD_DEG,), _f32),
        ],
    )(dst2)


def _sc_prop(tbl, src2, dst2, n_half):
    return pl.kernel(
        _make_sc_prop(n_half),
        out_type=jax.ShapeDtypeStruct((NC, n_half, NPAD_DEG, D_IN), _f32),
        mesh=_sc_mesh(),
        scratch_types=[
            pltpu.VMEM((GCH, ECH), jnp.int32),
            pltpu.VMEM((GCH, ECH), jnp.int32),
            pltpu.VMEM((ECH, D_IN), _f32),
            pltpu.VMEM((ECH, D_IN), _f32),
            pltpu.VMEM_SHARED((ACC_ROWS, D_IN), _f32),
            pltpu.SemaphoreType.DMA,
            pltpu.SemaphoreType.DMA,
            pltpu.SemaphoreType.DMA,
            pltpu.SemaphoreType.DMA,
        ],
    )(tbl, src2, dst2)


def _tc_weights_body(wi0, wn0, wi1, wn1, wf, bi0, bn0, bi1, bn1, bf,
                     wa0, wb0, wa1, wb1, b0, b1):
    wft = wf[pl.ds(0, D_OUT), :]
    wfb = wf[pl.ds(D_OUT, D_OUT), :]
    dot = functools.partial(jnp.dot, preferred_element_type=_f32)
    wa0[...] = dot(wi0[...], wft)
    wb0[...] = dot(wn0[...], wfb)
    wa1[...] = dot(wi1[...], wft)
    wb1[...] = dot(wn1[...], wfb)
    b0[...] = dot(bi0[...], wft) + dot(bn0[...], wfb) + bf[...]
    b1[...] = dot(bi1[...], wft) + dot(bn1[...], wfb) + bf[...]


def _tc_weights(wi0, wn0, wi1, wn1, wf, bi0, bn0, bi1, bn1, bf):
    outs = [
        jax.ShapeDtypeStruct((D_IN, D_OUT), _f32),
        jax.ShapeDtypeStruct((D_IN, D_OUT), _f32),
        jax.ShapeDtypeStruct((D_HID, D_OUT), _f32),
        jax.ShapeDtypeStruct((D_HID, D_OUT), _f32),
        jax.ShapeDtypeStruct((1, D_OUT), _f32),
        jax.ShapeDtypeStruct((1, D_OUT), _f32),
    ]
    return pl.pallas_call(_tc_weights_body, out_shape=outs)(
        wi0, wn0, wi1, wn1, wf, bi0, bn0, bi1, bn1, bf)


def _tc_prep_g_body(deg_ref, x_ref, dinv_ref, g0_ref):
    dv = lax.rsqrt(deg_ref[...] + 1.0)
    dinv_ref[...] = dv
    xv = x_ref[...]
    g0_ref[...] = jnp.stack([xv * dv[0], xv * dv[1]])[:, None]


def _tc_prep_g(deg3, x_pad):
    outs = [
        jax.ShapeDtypeStruct((NC, NPAD_DEG, 1), _f32),
        jax.ShapeDtypeStruct((NC, 1, NPAD_DEG, D_IN), _f32),
    ]
    return pl.pallas_call(
        _tc_prep_g_body,
        grid=(GRID_R,),
        in_specs=[
            pl.BlockSpec((NC, RB, 1), lambda r: (0, r, 0)),
            pl.BlockSpec((RB, D_IN), lambda r: (r, 0)),
        ],
        out_specs=[
            pl.BlockSpec((NC, RB, 1), lambda r: (0, r, 0)),
            pl.BlockSpec((NC, 1, RB, D_IN), lambda r: (0, 0, r, 0)),
        ],
        out_shape=outs,
    )(deg3, x_pad)


def _tc_dense1_body(s0_ref, dinv_ref, wa_ref, wb_ref, b_ref, g1_ref):
    s = s0_ref[...]
    dv = dinv_ref[...]
    pa = s[0, 0] * dv[0]
    pb = s[1, 0] * dv[1]
    dot = functools.partial(jnp.dot, preferred_element_type=_f32)
    t = jnp.maximum(dot(pa, wa_ref[...]) + dot(pb, wb_ref[...]) + b_ref[...], 0.0)
    ga = t * dv[0]
    gb = t * dv[1]
    g1_ref[...] = jnp.stack([
        jnp.stack([ga[:, :D_IN], ga[:, D_IN:]]),
        jnp.stack([gb[:, :D_IN], gb[:, D_IN:]]),
    ])


def _tc_dense1(s0, dinv, wa0, wb0, b0):
    return pl.pallas_call(
        _tc_dense1_body,
        grid=(GRID_R,),
        in_specs=[
            pl.BlockSpec((NC, 1, RB, D_IN), lambda r: (0, 0, r, 0)),
            pl.BlockSpec((NC, RB, 1), lambda r: (0, r, 0)),
            pl.BlockSpec((D_IN, D_OUT), lambda r: (0, 0)),
            pl.BlockSpec((D_IN, D_OUT), lambda r: (0, 0)),
            pl.BlockSpec((1, D_OUT), lambda r: (0, 0)),
        ],
        out_specs=pl.BlockSpec((NC, 2, RB, D_IN), lambda r: (0, 0, r, 0)),
        out_shape=jax.ShapeDtypeStruct((NC, 2, NPAD_DEG, D_IN), _f32),
    )(s0, dinv, wa0, wb0, b0)


def _tc_dense2_body(s1_ref, dinv_ref, wa_ref, wb_ref, b_ref, out_ref):
    s = s1_ref[...]
    dv = dinv_ref[...]
    pa = jnp.concatenate([s[0, 0], s[0, 1]], axis=1) * dv[0]
    pb = jnp.concatenate([s[1, 0], s[1, 1]], axis=1) * dv[1]
    dot = functools.partial(jnp.dot, preferred_element_type=_f32)
    z = jnp.maximum(dot(pa, wa_ref[...]) + dot(pb, wb_ref[...]) + b_ref[...], 0.0)
    r = pl.program_id(0)
    rows = r * RB + lax.broadcasted_iota(jnp.int32, (RB, 1), 0)
    zm = jnp.where(rows < N_NODES, z, 0.0)
    part = jnp.sum(zm, axis=0, keepdims=True)

    @pl.when(r == 0)
    def _():
        out_ref[...] = jnp.zeros_like(out_ref)

    out_ref[...] += part


def _tc_dense2(s1, dinv, wa1, wb1, b1):
    return pl.pallas_call(
        _tc_dense2_body,
        grid=(GRID_R,),
        in_specs=[
            pl.BlockSpec((NC, 2, RB, D_IN), lambda r: (0, 0, r, 0)),
            pl.BlockSpec((NC, RB, 1), lambda r: (0, r, 0)),
            pl.BlockSpec((D_HID, D_OUT), lambda r: (0, 0)),
            pl.BlockSpec((D_HID, D_OUT), lambda r: (0, 0)),
            pl.BlockSpec((1, D_OUT), lambda r: (0, 0)),
        ],
        out_specs=pl.BlockSpec((1, D_OUT), lambda r: (0, 0)),
        out_shape=jax.ShapeDtypeStruct((1, D_OUT), _f32),
    )(s1, dinv, wa1, wb1, b1)


def _pad_edges(ei):
    src = ei[0].astype(jnp.int32)
    dst = ei[1].astype(jnp.int32)
    pad = EPAD - N_EDGES
    src = jnp.concatenate([src, jnp.zeros((pad,), jnp.int32)])
    dst = jnp.concatenate([dst, jnp.full((pad,), N_NODES, jnp.int32)])
    return (src.reshape(NS, GROUPS, GCH, ECH), dst.reshape(NS, GROUPS, GCH, ECH))


def kernel(x, intra_edge_index, inter_edge_index,
           W_intra0, b_intra0, W_inter0, b_inter0,
           W_intra1, b_intra1, W_inter1, b_inter1,
           W_fus, b_fus):
    src_a, dst_a = _pad_edges(intra_edge_index)
    src_b, dst_b = _pad_edges(inter_edge_index)
    src2 = jnp.stack([src_a, src_b])
    dst2 = jnp.stack([dst_a, dst_b])

    deg = _sc_deg(dst2)
    deg3 = deg[:, :, None]

    x_pad = jnp.concatenate(
        [x, jnp.zeros((NPAD_DEG - N_NODES, D_IN), _f32)], axis=0)

    wa0, wb0, wa1, wb1, b0, b1 = _tc_weights(
        W_intra0, W_inter0, W_intra1, W_inter1, W_fus,
        b_intra0[None, :], b_inter0[None, :],
        b_intra1[None, :], b_inter1[None, :], b_fus[None, :])

    dinv, g0 = _tc_prep_g(deg3, x_pad)

    s0 = _sc_prop(g0, src2, dst2, 1)
    g1 = _tc_dense1(s0, dinv, wa0, wb0, b0)
    s1 = _sc_prop(g1, src2, dst2, 2)
    return _tc_dense2(s1, dinv, wa1, wb1, b1)
